# baseline (speedup 1.0000x reference)
import functools

import jax
import jax.numpy as jnp
from jax import lax
from jax.experimental import pallas as pl
from jax.experimental.pallas import tpu as pltpu
from jax.experimental.pallas import tpu_sc as plsc

N_NODES = 10000
NUM_ATOM_TYPE = 119
MASK_RATIO = 0.15
NOISE_VAL = 0.1
EPS = 1e-5

_BE = 2000
_NM = 1500
_NMP = 1504


def _edge_chain_body(atoms_ref, ea_ref, pe2_ref, pn2_ref, hp_ref, a_ref,
                     b_ref, wdT_ref, bd_ref, g_ref, lb_ref, wout_ref,
                     bout_ref, out_ref):
    atoms = atoms_ref[...]
    cols = jax.lax.broadcasted_iota(jnp.int32, (_BE, 128), 1)
    onehot = jnp.where(cols == atoms, 1.0, 0.0)
    hp_src = jnp.dot(onehot, hp_ref[...], preferred_element_type=jnp.float32)
    pn = jnp.sqrt(pn2_ref[...])
    q = jnp.dot(ea_ref[...], a_ref[0:4, :],
                preferred_element_type=jnp.float32) + pn * b_ref[...]
    pe = jnp.maximum(hp_src + q, 0.0)
    t = jnp.dot(pe, wdT_ref[...],
                preferred_element_type=jnp.float32) + bd_ref[...]
    t = jax.nn.gelu(t)
    mu = jnp.mean(t, axis=-1, keepdims=True)
    var = jnp.mean((t - mu) * (t - mu), axis=-1, keepdims=True)
    tn = (t - mu) * jax.lax.rsqrt(var + EPS) * g_ref[...] + lb_ref[...]
    dout = jnp.sum(tn * wout_ref[...], axis=-1, keepdims=True) + bout_ref[0, 0]
    out_ref[...] = dout - jnp.sqrt(pe2_ref[...])


def _edge_chain(atoms, edge_attr, pe2, pn2, hp_pad, A, b_row, dh_dense_w,
                dh_dense_b, dh_ln_g, dh_ln_b, dh_out_w, dh_out_b):
    E = atoms.shape[0]
    D = 128
    grid = (E // _BE,)
    full = lambda i: (0, 0)
    blk = lambda i: (i, 0)
    return pl.pallas_call(
        _edge_chain_body,
        grid=grid,
        in_specs=[
            pl.BlockSpec((_BE, 1), blk),
            pl.BlockSpec((_BE, 4), blk),
            pl.BlockSpec((_BE, 1), blk),
            pl.BlockSpec((_BE, 1), blk),
            pl.BlockSpec((D, D), full),
            pl.BlockSpec((8, D), full),
            pl.BlockSpec((1, D), full),
            pl.BlockSpec((D, D), full),
            pl.BlockSpec((1, D), full),
            pl.BlockSpec((1, D), full),
            pl.BlockSpec((1, D), full),
            pl.BlockSpec((1, D), full),
            pl.BlockSpec((1, 1), full),
        ],
        out_specs=pl.BlockSpec((_BE, 1), blk),
        out_shape=jax.ShapeDtypeStruct((E, 1), jnp.float32),
    )(atoms.reshape(E, 1), edge_attr, pe2.reshape(E, 1), pn2.reshape(E, 1),
      hp_pad, A, b_row, dh_dense_w.T, dh_dense_b.reshape(1, D),
      dh_ln_g.reshape(1, D), dh_ln_b.reshape(1, D), dh_out_w.reshape(1, D),
      dh_out_b.reshape(1, 1))


def _node_head_body(cg_ref, eam_ref, pnm_ref, snm_ref, wemb_ref, wedge_ref,
                    wpe_ref, wgnn_ref, mdwT_ref, mdb_ref, mg_ref, mb_ref,
                    mwT_ref, mbias_ref, tgt_ref, out_ref):
    C = cg_ref[0] + cg_ref[1]
    Ea = eam_ref[0] + eam_ref[1]
    pn = pnm_ref[0] + pnm_ref[1]
    G = jnp.dot(C, wemb_ref[...], preferred_element_type=jnp.float32)
    EaW = jnp.dot(Ea[:, 0:4], wedge_ref[0:4, :],
                  preferred_element_type=jnp.float32)
    aggm = G + EaW + pn * wpe_ref[...]
    h0m = wemb_ref[NUM_ATOM_TYPE:NUM_ATOM_TYPE + 1, :]
    z = jnp.dot(h0m + aggm * snm_ref[...], wgnn_ref[...],
                preferred_element_type=jnp.float32)
    feats = jnp.maximum(z, 0.0)
    h = jnp.dot(feats, mdwT_ref[...],
                preferred_element_type=jnp.float32) + mdb_ref[...]
    h = jax.nn.gelu(h)
    mu = jnp.mean(h, axis=-1, keepdims=True)
    var = jnp.mean((h - mu) * (h - mu), axis=-1, keepdims=True)
    h = (h - mu) * jax.lax.rsqrt(var + EPS) * mg_ref[...] + mb_ref[...]
    pred = jnp.dot(h, mwT_ref[...],
                   preferred_element_type=jnp.float32) + mbias_ref[...]
    cols = jax.lax.broadcasted_iota(jnp.int32, (_NMP, 128), 1)
    rows = jax.lax.broadcasted_iota(jnp.int32, (_NMP, 128), 0)
    pred = jnp.where(cols < NUM_ATOM_TYPE, pred, -1e30)
    mx = jnp.max(pred, axis=-1, keepdims=True)
    lse = mx + jnp.log(jnp.sum(jnp.exp(pred - mx), axis=-1, keepdims=True))
    logp = pred - lse
    sel = (cols == tgt_ref[...]) & (rows < _NM)
    out_ref[...] = (-jnp.sum(jnp.where(sel, logp, 0.0)) / _NM).reshape(1, 1)


def _node_head(cg, eam, pnm, snm, wemb_pad, wedge_pad, w_pe, W_gnn,
               mlm_dense_w, mlm_dense_b, mlm_ln_g, mlm_ln_b, mlm_weightT_pad,
               mlm_bias_pad, tgt):
    D = 128
    full2 = lambda: None
    specs = [
        pl.BlockSpec((2, _NMP, D), lambda: (0, 0, 0)),
        pl.BlockSpec((2, _NMP, 8), lambda: (0, 0, 0)),
        pl.BlockSpec((2, _NMP, 1), lambda: (0, 0, 0)),
        pl.BlockSpec((_NMP, 1), lambda: (0, 0)),
        pl.BlockSpec((D, D), lambda: (0, 0)),
        pl.BlockSpec((8, D), lambda: (0, 0)),
        pl.BlockSpec((1, D), lambda: (0, 0)),
        pl.BlockSpec((D, D), lambda: (0, 0)),
        pl.BlockSpec((D, D), lambda: (0, 0)),
        pl.BlockSpec((1, D), lambda: (0, 0)),
        pl.BlockSpec((1, D), lambda: (0, 0)),
        pl.BlockSpec((1, D), lambda: (0, 0)),
        pl.BlockSpec((D, D), lambda: (0, 0)),
        pl.BlockSpec((1, D), lambda: (0, 0)),
        pl.BlockSpec((_NMP, 1), lambda: (0, 0)),
    ]
    return pl.pallas_call(
        _node_head_body,
        in_specs=specs,
        out_specs=pl.BlockSpec((1, 1), lambda: (0, 0)),
        out_shape=jax.ShapeDtypeStruct((1, 1), jnp.float32),
    )(cg, eam, pnm, snm, wemb_pad, wedge_pad, w_pe, W_gnn,
      mlm_dense_w.T, mlm_dense_b.reshape(1, D), mlm_ln_g.reshape(1, D),
      mlm_ln_b.reshape(1, D), mlm_weightT_pad, mlm_bias_pad,
      tgt.reshape(_NMP, 1))


_SLOTS_PER_CORE = 750
_PASS_SLOTS = 22
_NPASS = 35
_W = _PASS_SLOTS * N_NODES
_TOT = 56 * 4096
_CH = 2000


def _dedup_sc(src, dst, v, lut):
    E = src.shape[0]
    EPW = E // 16
    NCH = EPW // _CH
    CAP = 2 * EPW + 128
    mesh = plsc.VectorSubcoreMesh(core_axis_name="c", subcore_axis_name="s")

    @functools.partial(
        pl.kernel, mesh=mesh,
        out_type=jax.ShapeDtypeStruct((32, 32), jnp.float32),
        compiler_params=pltpu.CompilerParams(needs_layout_passes=False),
        scratch_types=[
            pltpu.VMEM((N_NODES,), jnp.int32),
            pltpu.VMEM((_CH,), jnp.int32),
            pltpu.VMEM((_CH,), jnp.int32),
            pltpu.VMEM((_CH,), jnp.float32),
            pltpu.VMEM((CAP,), jnp.int32),
            pltpu.VMEM((CAP,), jnp.float32),
            pltpu.VMEM((4096,), jnp.float32),
            pltpu.VMEM((144,), jnp.int32),
            pltpu.VMEM((144,), jnp.float32),
            pltpu.VMEM((128,), jnp.float32),
            pltpu.VMEM((128,), jnp.float32),
            pltpu.VMEM((128,), jnp.float32),
            pltpu.VMEM((128,), jnp.float32),
            pltpu.VMEM((128,), jnp.int32),
            pltpu.VMEM((32,), jnp.float32),
            pltpu.VMEM_SHARED((_TOT,), jnp.float32),
            pltpu.VMEM_SHARED((_TOT,), jnp.float32),
            pltpu.SemaphoreType.DMA,
        ])
    def k(src_h, dst_h, v_h, lut_h, out_h, lut_v, srcb, dstb, vb, cellb,
          valb, zbuf, gidx, gval, ones_v, zeros_v, rsv, rsc, sidx, obuf,
          svals, scnt, sem):
        c = lax.axis_index("c")
        s = lax.axis_index("s")
        wid = s * 2 + c
        iota = lax.iota(jnp.int32, 16)
        zero16 = jnp.zeros((16,), jnp.float32)
        one16 = jnp.ones((16,), jnp.float32)

        pltpu.sync_copy(lut_h, lut_v)

        def zb_loop(i, carry):
            zbuf[pl.ds(i * 16, 16)] = zero16
            return carry
        lax.fori_loop(0, 256, zb_loop, 0)
        for j in range(8):
            ones_v[pl.ds(j * 16, 16)] = one16
            zeros_v[pl.ds(j * 16, 16)] = zero16

        def zs_loop(kk, carry):
            blk = kk * 16 + s

            @pl.when(blk < _TOT // 4096)
            def _():
                pltpu.sync_copy(zbuf, svals.at[pl.ds(blk * 4096, 4096)])
                pltpu.sync_copy(zbuf, scnt.at[pl.ds(blk * 4096, 4096)])
            return carry
        lax.fori_loop(0, (_TOT // 4096 + 15) // 16, zs_loop, 0)

        slot_lo = c * _SLOTS_PER_CORE

        def build_chunk(ch, n):
            base = s * EPW + ch * _CH
            pltpu.sync_copy(src_h.at[pl.ds(base, _CH)], srcb)
            pltpu.sync_copy(dst_h.at[pl.ds(base, _CH)], dstb)
            pltpu.sync_copy(v_h.at[pl.ds(base, _CH)], vb)

            def vloop(i, n):
                s16 = srcb[pl.ds(i * 16, 16)]
                d16 = dstb[pl.ds(i * 16, 16)]
                v16 = vb[pl.ds(i * 16, 16)]
                ls = plsc.load_gather(lut_v, [s16]) - slot_lo
                ld = plsc.load_gather(lut_v, [d16]) - slot_lo
                mA = (ls >= 0) & (ls < _SLOTS_PER_CORE)
                cellA = ls * N_NODES + d16
                plsc.store_compressed(cellb.at[pl.ds(n, 16)], cellA, mask=mA)
                plsc.store_compressed(valb.at[pl.ds(n, 16)], v16, mask=mA)
                n = n + jnp.sum(mA.astype(jnp.int32))
                mB = (ld >= 0) & (ld < _SLOTS_PER_CORE)
                cellB = ld * N_NODES + s16
                plsc.store_compressed(cellb.at[pl.ds(n, 16)], cellB, mask=mB)
                plsc.store_compressed(valb.at[pl.ds(n, 16)], v16, mask=mB)
                n = n + jnp.sum(mB.astype(jnp.int32))
                return n
            return lax.fori_loop(0, _CH // 16, vloop, n)
        n_inst = lax.fori_loop(0, NCH, build_chunk, jnp.int32(0))
        plsc.subcore_barrier()

        ng = (n_inst + 127) // 128

        def fill_group(g, p, with_val):
            lo_cell = p * _W
            off = jnp.int32(0)
            for j in range(8):
                base_i = g * 128 + j * 16
                cells16 = cellb[pl.ds(base_i, 16)]
                valid = (base_i + iota) < n_inst
                inr = valid & (cells16 >= lo_cell) & (cells16 < lo_cell + _W)
                loc = cells16 - lo_cell
                plsc.store_compressed(gidx.at[pl.ds(off, 16)], loc, mask=inr)
                if with_val:
                    v16 = valb[pl.ds(base_i, 16)]
                    plsc.store_compressed(gval.at[pl.ds(off, 16)], v16, mask=inr)
                off = off + jnp.sum(inr.astype(jnp.int32))
            dump = _W + iota
            for j in range(8):
                sel = (j * 16 + iota) < off
                row = gidx[pl.ds(j * 16, 16)]
                gidx[pl.ds(j * 16, 16)] = jnp.where(sel, row, dump)
                if with_val:
                    rv = gval[pl.ds(j * 16, 16)]
                    gval[pl.ds(j * 16, 16)] = jnp.where(sel, rv, 0.0)
            for j in range(8):
                sidx[pl.ds(j * 16, 16)] = gidx[pl.ds(j * 16, 16)]

        def pass_body(p, carry):
            num16, den16 = carry

            def g_scatter(g, carry2):
                fill_group(g, p, True)
                for j in range(8):
                    rsv[pl.ds(j * 16, 16)] = gval[pl.ds(j * 16, 16)]
                pltpu.sync_copy(rsv, svals.at[sidx], add=True)
                pltpu.sync_copy(ones_v, scnt.at[sidx], add=True)
                return carry2
            lax.fori_loop(0, ng, g_scatter, 0)
            plsc.subcore_barrier()

            def g_read(g, carry2):
                num16, den16 = carry2
                fill_group(g, p, False)
                pltpu.async_copy(svals.at[sidx], rsv, sem).wait()
                pltpu.async_copy(scnt.at[sidx], rsc, sem).wait()
                for j in range(8):
                    sj = rsv[pl.ds(j * 16, 16)]
                    cj = rsc[pl.ds(j * 16, 16)]
                    ij = sidx[pl.ds(j * 16, 16)]
                    real = ij < _W
                    cs = jnp.maximum(cj, 1.0)
                    r = sj / cs
                    ar = jnp.abs(r)
                    hb = jnp.where(ar < 1.0, 0.5 * r * r, ar - 0.5)
                    num16 = num16 + jnp.where(real, hb / cs, 0.0)
                    den16 = den16 + jnp.where(real, 1.0 / cs, 0.0)
                return (num16, den16)
            num16, den16 = lax.fori_loop(0, ng, g_read, (num16, den16))
            plsc.subcore_barrier()

            def g_reset(g, carry2):
                fill_group(g, p, False)
                pltpu.sync_copy(zeros_v, svals.at[sidx])
                pltpu.sync_copy(zeros_v, scnt.at[sidx])
                return carry2
            lax.fori_loop(0, ng, g_reset, 0)
            plsc.subcore_barrier()
            return (num16, den16)

        num16, den16 = lax.fori_loop(0, _NPASS, pass_body, (zero16, zero16))

        obuf[pl.ds(0, 16)] = num16
        obuf[pl.ds(16, 16)] = den16
        pltpu.sync_copy(obuf, out_h.at[wid])

    return k(src, dst, v, lut)


def _huber(x):
    ax = jnp.abs(x)
    return jnp.where(ax < 1.0, 0.5 * x * x, ax - 0.5)


def kernel(x, edge_index, edge_attr, snorm_n, EigVals, EigVecs, W_embed,
           W_edge, w_pe, W_gnn, W_pe2, mlm_dense_w, mlm_dense_b, mlm_ln_g,
           mlm_ln_b, mlm_weight, mlm_bias, dh_dense_w, dh_dense_b, dh_ln_g,
           dh_ln_b, dh_out_w, dh_out_b):
    N = x.shape[0]
    E = edge_index.shape[1]
    u = jnp.nan_to_num(EigVecs)
    src = edge_index[0]
    dst = edge_index[1]

    mkey = jax.random.key(42)
    perm = jax.random.permutation(mkey, N)
    num_mask = int(MASK_RATIO * N)
    mask_nodes = perm[:num_mask]
    noise = NOISE_VAL * jax.random.normal(
        jax.random.fold_in(mkey, 1), (num_mask, u.shape[1]),
        dtype=jnp.float32)
    node_is_masked = jnp.zeros((N,), bool).at[mask_nodes].set(True)
    lut = jnp.full((N,), -1, jnp.int32).at[mask_nodes].set(
        jnp.arange(num_mask, dtype=jnp.int32))

    u_masked = u.at[mask_nodes].add(noise)
    xm0 = jnp.where(node_is_masked, NUM_ATOM_TYPE, x[:, 0]).astype(jnp.int32)

    wemb_pad = jnp.zeros((128, 128), jnp.float32).at[:NUM_ATOM_TYPE + 1].set(
        W_embed)
    hp_pad = wemb_pad @ W_pe2
    A = jnp.zeros((8, 128), jnp.float32).at[:4].set(W_edge @ W_pe2)
    b_row = w_pe @ W_pe2
    wedge_pad = jnp.zeros((8, 128), jnp.float32).at[:4].set(W_edge)
    mlm_wT_pad = jnp.zeros((128, 128), jnp.float32).at[:, :NUM_ATOM_TYPE].set(
        mlm_weight.T)
    mlm_bias_pad = jnp.zeros((1, 128), jnp.float32).at[0, :NUM_ATOM_TYPE].set(
        mlm_bias)

    atoms = xm0[src]
    du = u[src] - u[dst]
    pe2 = jnp.sum(du * du, axis=-1)
    dun = u_masked[src] - u_masked[dst]
    pn2 = jnp.sum(dun * dun, axis=-1)

    sdst = lut[dst]
    mm = sdst >= 0
    idx_safe = jnp.where(mm, sdst, _NM)
    cg0 = jnp.zeros((_NMP, 128), jnp.float32).at[idx_safe, atoms].add(
        jnp.where(mm, 1.0, 0.0))
    eam0 = jnp.zeros((_NMP, 8), jnp.float32).at[idx_safe, :4].add(
        jnp.where(mm, 1.0, 0.0)[:, None] * edge_attr)
    pnm0 = jnp.zeros((_NMP, 1), jnp.float32).at[idx_safe, 0].add(
        jnp.where(mm, jnp.sqrt(pn2), 0.0))
    cg = jnp.stack([cg0, jnp.zeros_like(cg0)])
    eam = jnp.stack([eam0, jnp.zeros_like(eam0)])
    pnm = jnp.stack([pnm0, jnp.zeros_like(pnm0)])

    snm = jnp.zeros((_NMP, 1), jnp.float32).at[:num_mask, 0].set(
        snorm_n[mask_nodes, 0])
    tgt = jnp.zeros((_NMP,), jnp.int32).at[:num_mask].set(x[mask_nodes, 0])

    v = _edge_chain(atoms, edge_attr, pe2, pn2, hp_pad, A, b_row, dh_dense_w,
                    dh_dense_b, dh_ln_g, dh_ln_b, dh_out_w, dh_out_b)[:, 0]
    atom_loss = _node_head(cg, eam, pnm, snm, wemb_pad, wedge_pad, w_pe,
                           W_gnn, mlm_dense_w, mlm_dense_b, mlm_ln_g,
                           mlm_ln_b, mlm_wT_pad, mlm_bias_pad, tgt)[0, 0]

    nd = _dedup_sc(src, dst, v, lut)
    num = jnp.sum(nd[:, :16])
    den = jnp.sum(nd[:, 16:])
    pe_loss = num / den

    return atom_loss + pe_loss


# speedup vs baseline: 59.2158x; 59.2158x over previous
import functools

import jax
import jax.numpy as jnp
from jax import lax
from jax.experimental import pallas as pl
from jax.experimental.pallas import tpu as pltpu
from jax.experimental.pallas import tpu_sc as plsc

N_NODES = 10000
NUM_ATOM_TYPE = 119
MASK_RATIO = 0.15
NOISE_VAL = 0.1
EPS = 1e-5

_BE = 2000
_NM = 1500
_NMP = 1504


def _edge_chain_body(atoms_ref, ea_ref, pe2_ref, pn2_ref, hp_ref, a_ref,
                     b_ref, wdT_ref, bd_ref, g_ref, lb_ref, wout_ref,
                     bout_ref, out_ref):
    atoms = atoms_ref[...]
    cols = jax.lax.broadcasted_iota(jnp.int32, (_BE, 128), 1)
    onehot = jnp.where(cols == atoms, 1.0, 0.0)
    hp_src = jnp.dot(onehot, hp_ref[...], preferred_element_type=jnp.float32)
    pn = jnp.sqrt(pn2_ref[...])
    q = jnp.dot(ea_ref[...], a_ref[0:4, :],
                preferred_element_type=jnp.float32) + pn * b_ref[...]
    pe = jnp.maximum(hp_src + q, 0.0)
    t = jnp.dot(pe, wdT_ref[...],
                preferred_element_type=jnp.float32) + bd_ref[...]
    t = jax.nn.gelu(t)
    mu = jnp.mean(t, axis=-1, keepdims=True)
    var = jnp.mean((t - mu) * (t - mu), axis=-1, keepdims=True)
    tn = (t - mu) * jax.lax.rsqrt(var + EPS) * g_ref[...] + lb_ref[...]
    dout = jnp.sum(tn * wout_ref[...], axis=-1, keepdims=True) + bout_ref[0, 0]
    out_ref[...] = dout - jnp.sqrt(pe2_ref[...])


def _edge_chain(atoms, edge_attr, pe2, pn2, hp_pad, A, b_row, dh_dense_w,
                dh_dense_b, dh_ln_g, dh_ln_b, dh_out_w, dh_out_b):
    E = atoms.shape[0]
    D = 128
    grid = (E // _BE,)
    full = lambda i: (0, 0)
    blk = lambda i: (i, 0)
    return pl.pallas_call(
        _edge_chain_body,
        grid=grid,
        in_specs=[
            pl.BlockSpec((_BE, 1), blk),
            pl.BlockSpec((_BE, 4), blk),
            pl.BlockSpec((_BE, 1), blk),
            pl.BlockSpec((_BE, 1), blk),
            pl.BlockSpec((D, D), full),
            pl.BlockSpec((8, D), full),
            pl.BlockSpec((1, D), full),
            pl.BlockSpec((D, D), full),
            pl.BlockSpec((1, D), full),
            pl.BlockSpec((1, D), full),
            pl.BlockSpec((1, D), full),
            pl.BlockSpec((1, D), full),
            pl.BlockSpec((1, 1), full),
        ],
        out_specs=pl.BlockSpec((_BE, 1), blk),
        out_shape=jax.ShapeDtypeStruct((E, 1), jnp.float32),
    )(atoms.reshape(E, 1), edge_attr, pe2.reshape(E, 1), pn2.reshape(E, 1),
      hp_pad, A, b_row, dh_dense_w.T, dh_dense_b.reshape(1, D),
      dh_ln_g.reshape(1, D), dh_ln_b.reshape(1, D), dh_out_w.reshape(1, D),
      dh_out_b.reshape(1, 1))


def _node_head_body(cg_ref, eam_ref, pnm_ref, snm_ref, wemb_ref, wedge_ref,
                    wpe_ref, wgnn_ref, mdwT_ref, mdb_ref, mg_ref, mb_ref,
                    mwT_ref, mbias_ref, tgt_ref, out_ref):
    C = cg_ref[0] + cg_ref[1]
    Ea = eam_ref[0] + eam_ref[1]
    pn = pnm_ref[0] + pnm_ref[1]
    G = jnp.dot(C, wemb_ref[...], preferred_element_type=jnp.float32)
    EaW = jnp.dot(Ea[:, 0:4], wedge_ref[0:4, :],
                  preferred_element_type=jnp.float32)
    aggm = G + EaW + pn * wpe_ref[...]
    h0m = wemb_ref[NUM_ATOM_TYPE:NUM_ATOM_TYPE + 1, :]
    z = jnp.dot(h0m + aggm * snm_ref[...], wgnn_ref[...],
                preferred_element_type=jnp.float32)
    feats = jnp.maximum(z, 0.0)
    h = jnp.dot(feats, mdwT_ref[...],
                preferred_element_type=jnp.float32) + mdb_ref[...]
    h = jax.nn.gelu(h)
    mu = jnp.mean(h, axis=-1, keepdims=True)
    var = jnp.mean((h - mu) * (h - mu), axis=-1, keepdims=True)
    h = (h - mu) * jax.lax.rsqrt(var + EPS) * mg_ref[...] + mb_ref[...]
    pred = jnp.dot(h, mwT_ref[...],
                   preferred_element_type=jnp.float32) + mbias_ref[...]
    cols = jax.lax.broadcasted_iota(jnp.int32, (_NMP, 128), 1)
    rows = jax.lax.broadcasted_iota(jnp.int32, (_NMP, 128), 0)
    pred = jnp.where(cols < NUM_ATOM_TYPE, pred, -1e30)
    mx = jnp.max(pred, axis=-1, keepdims=True)
    lse = mx + jnp.log(jnp.sum(jnp.exp(pred - mx), axis=-1, keepdims=True))
    logp = pred - lse
    sel = (cols == tgt_ref[...]) & (rows < _NM)
    out_ref[...] = (-jnp.sum(jnp.where(sel, logp, 0.0)) / _NM).reshape(1, 1)


def _node_head(cg, eam, pnm, snm, wemb_pad, wedge_pad, w_pe, W_gnn,
               mlm_dense_w, mlm_dense_b, mlm_ln_g, mlm_ln_b, mlm_weightT_pad,
               mlm_bias_pad, tgt):
    D = 128
    full2 = lambda: None
    specs = [
        pl.BlockSpec((2, _NMP, D), lambda: (0, 0, 0)),
        pl.BlockSpec((2, _NMP, 8), lambda: (0, 0, 0)),
        pl.BlockSpec((2, _NMP, 1), lambda: (0, 0, 0)),
        pl.BlockSpec((_NMP, 1), lambda: (0, 0)),
        pl.BlockSpec((D, D), lambda: (0, 0)),
        pl.BlockSpec((8, D), lambda: (0, 0)),
        pl.BlockSpec((1, D), lambda: (0, 0)),
        pl.BlockSpec((D, D), lambda: (0, 0)),
        pl.BlockSpec((D, D), lambda: (0, 0)),
        pl.BlockSpec((1, D), lambda: (0, 0)),
        pl.BlockSpec((1, D), lambda: (0, 0)),
        pl.BlockSpec((1, D), lambda: (0, 0)),
        pl.BlockSpec((D, D), lambda: (0, 0)),
        pl.BlockSpec((1, D), lambda: (0, 0)),
        pl.BlockSpec((_NMP, 1), lambda: (0, 0)),
    ]
    return pl.pallas_call(
        _node_head_body,
        in_specs=specs,
        out_specs=pl.BlockSpec((1, 1), lambda: (0, 0)),
        out_shape=jax.ShapeDtypeStruct((1, 1), jnp.float32),
    )(cg, eam, pnm, snm, wemb_pad, wedge_pad, w_pe, W_gnn,
      mlm_dense_w.T, mlm_dense_b.reshape(1, D), mlm_ln_g.reshape(1, D),
      mlm_ln_b.reshape(1, D), mlm_weightT_pad, mlm_bias_pad,
      tgt.reshape(_NMP, 1))


_SLOTS_PER_CORE = 750
_PASS_SLOTS = 17
_NPASS = 45
_W = _PASS_SLOTS * N_NODES
_TOT = 43 * 4096
_CH = 2000


def _dedup_sc(src, dst, v, lut):
    E = src.shape[0]
    EPW = E // 16
    NCH = EPW // _CH
    CAP = 2 * EPW + 128
    mesh = plsc.VectorSubcoreMesh(core_axis_name="c", subcore_axis_name="s")

    @functools.partial(
        pl.kernel, mesh=mesh,
        out_type=jax.ShapeDtypeStruct((32, 32), jnp.float32),
        compiler_params=pltpu.CompilerParams(needs_layout_passes=False),
        scratch_types=[
            pltpu.VMEM((N_NODES,), jnp.int32),
            pltpu.VMEM((_CH,), jnp.int32),
            pltpu.VMEM((_CH,), jnp.int32),
            pltpu.VMEM((_CH,), jnp.float32),
            pltpu.VMEM((CAP,), jnp.int32),
            pltpu.VMEM((CAP,), jnp.float32),
            pltpu.VMEM((4096,), jnp.float32),
            pltpu.VMEM((144,), jnp.int32),
            pltpu.VMEM((144,), jnp.float32),
            pltpu.VMEM((128,), jnp.float32),
            pltpu.VMEM((128,), jnp.float32),
            pltpu.VMEM((128,), jnp.float32),
            pltpu.VMEM((128,), jnp.float32),
            pltpu.VMEM((128,), jnp.int32),
            pltpu.VMEM((32,), jnp.float32),
            pltpu.VMEM_SHARED((_TOT,), jnp.float32),
            pltpu.VMEM_SHARED((_TOT,), jnp.float32),
            pltpu.SemaphoreType.DMA,
        ])
    def k(src_h, dst_h, v_h, lut_h, out_h, lut_v, srcb, dstb, vb, cellb,
          valb, zbuf, gidx, gval, ones_v, zeros_v, rsv, rsc, sidx, obuf,
          svals, scnt, sem):
        c = lax.axis_index("c")
        s = lax.axis_index("s")
        wid = s * 2 + c
        iota = lax.iota(jnp.int32, 16)
        zero16 = jnp.zeros((16,), jnp.float32)
        one16 = jnp.ones((16,), jnp.float32)

        pltpu.sync_copy(lut_h, lut_v)

        def zb_loop(i, carry):
            zbuf[pl.ds(i * 16, 16)] = zero16
            return carry
        lax.fori_loop(0, 256, zb_loop, 0)
        for j in range(8):
            ones_v[pl.ds(j * 16, 16)] = one16
            zeros_v[pl.ds(j * 16, 16)] = zero16

        def zs_loop(kk, carry):
            blk = kk * 16 + s

            @pl.when(blk < _TOT // 4096)
            def _():
                pltpu.sync_copy(zbuf, svals.at[pl.ds(blk * 4096, 4096)])
                pltpu.sync_copy(zbuf, scnt.at[pl.ds(blk * 4096, 4096)])
            return carry
        lax.fori_loop(0, (_TOT // 4096 + 15) // 16, zs_loop, 0)

        slot_lo = c * _SLOTS_PER_CORE

        def build_chunk(ch, n):
            base = s * EPW + ch * _CH
            pltpu.sync_copy(src_h.at[pl.ds(base, _CH)], srcb)
            pltpu.sync_copy(dst_h.at[pl.ds(base, _CH)], dstb)
            pltpu.sync_copy(v_h.at[pl.ds(base, _CH)], vb)

            def vloop(i, n):
                s16 = srcb[pl.ds(i * 16, 16)]
                d16 = dstb[pl.ds(i * 16, 16)]
                v16 = vb[pl.ds(i * 16, 16)]
                ls = plsc.load_gather(lut_v, [s16]) - slot_lo
                ld = plsc.load_gather(lut_v, [d16]) - slot_lo
                mA = (ls >= 0) & (ls < _SLOTS_PER_CORE)
                cellA = ls * N_NODES + d16
                plsc.store_compressed(cellb.at[pl.ds(n, 16)], cellA, mask=mA)
                plsc.store_compressed(valb.at[pl.ds(n, 16)], v16, mask=mA)
                n = n + jnp.sum(mA.astype(jnp.int32))
                mB = (ld >= 0) & (ld < _SLOTS_PER_CORE)
                cellB = ld * N_NODES + s16
                plsc.store_compressed(cellb.at[pl.ds(n, 16)], cellB, mask=mB)
                plsc.store_compressed(valb.at[pl.ds(n, 16)], v16, mask=mB)
                n = n + jnp.sum(mB.astype(jnp.int32))
                return n
            return lax.fori_loop(0, _CH // 16, vloop, n)
        n_inst = lax.fori_loop(0, NCH, build_chunk, jnp.int32(0))
        plsc.subcore_barrier()

        ng = (n_inst + 127) // 128

        def fill_group(g, p, with_val):
            lo_cell = p * _W
            off = jnp.int32(0)
            for j in range(8):
                base_i = g * 128 + j * 16
                cells16 = cellb[pl.ds(base_i, 16)]
                valid = (base_i + iota) < n_inst
                inr = valid & (cells16 >= lo_cell) & (cells16 < lo_cell + _W)
                loc = cells16 - lo_cell
                plsc.store_compressed(gidx.at[pl.ds(off, 16)], loc, mask=inr)
                if with_val:
                    v16 = valb[pl.ds(base_i, 16)]
                    plsc.store_compressed(gval.at[pl.ds(off, 16)], v16, mask=inr)
                off = off + jnp.sum(inr.astype(jnp.int32))
            dump = _W + iota
            for j in range(8):
                sel = (j * 16 + iota) < off
                row = gidx[pl.ds(j * 16, 16)]
                gidx[pl.ds(j * 16, 16)] = jnp.where(sel, row, dump)
                if with_val:
                    rv = gval[pl.ds(j * 16, 16)]
                    gval[pl.ds(j * 16, 16)] = jnp.where(sel, rv, 0.0)
            for j in range(8):
                sidx[pl.ds(j * 16, 16)] = gidx[pl.ds(j * 16, 16)]

        def pass_body(p, carry):
            num16, den16 = carry

            def g_scatter(g, carry2):
                fill_group(g, p, True)
                for j in range(8):
                    rsv[pl.ds(j * 16, 16)] = gval[pl.ds(j * 16, 16)]
                pltpu.sync_copy(rsv, svals.at[sidx], add=True)
                pltpu.sync_copy(ones_v, scnt.at[sidx], add=True)
                return carry2
            lax.fori_loop(0, ng, g_scatter, 0)
            plsc.subcore_barrier()

            def g_read(g, carry2):
                num16, den16 = carry2
                fill_group(g, p, False)
                pltpu.async_copy(svals.at[sidx], rsv, sem).wait()
                pltpu.async_copy(scnt.at[sidx], rsc, sem).wait()
                for j in range(8):
                    sj = rsv[pl.ds(j * 16, 16)]
                    cj = rsc[pl.ds(j * 16, 16)]
                    ij = sidx[pl.ds(j * 16, 16)]
                    real = ij < _W
                    cs = jnp.maximum(cj, 1.0)
                    r = sj / cs
                    ar = jnp.abs(r)
                    hb = jnp.where(ar < 1.0, 0.5 * r * r, ar - 0.5)
                    num16 = num16 + jnp.where(real, hb / cs, 0.0)
                    den16 = den16 + jnp.where(real, 1.0 / cs, 0.0)
                return (num16, den16)
            num16, den16 = lax.fori_loop(0, ng, g_read, (num16, den16))
            plsc.subcore_barrier()

            def g_reset(g, carry2):
                fill_group(g, p, False)
                pltpu.sync_copy(zeros_v, svals.at[sidx])
                pltpu.sync_copy(zeros_v, scnt.at[sidx])
                return carry2
            lax.fori_loop(0, ng, g_reset, 0)
            plsc.subcore_barrier()
            return (num16, den16)

        num16, den16 = lax.fori_loop(0, _NPASS, pass_body, (zero16, zero16))

        obuf[pl.ds(0, 16)] = num16
        obuf[pl.ds(16, 16)] = den16
        pltpu.sync_copy(obuf, out_h.at[wid])

    return k(src, dst, v, lut)


def _huber(x):
    ax = jnp.abs(x)
    return jnp.where(ax < 1.0, 0.5 * x * x, ax - 0.5)


def kernel(x, edge_index, edge_attr, snorm_n, EigVals, EigVecs, W_embed,
           W_edge, w_pe, W_gnn, W_pe2, mlm_dense_w, mlm_dense_b, mlm_ln_g,
           mlm_ln_b, mlm_weight, mlm_bias, dh_dense_w, dh_dense_b, dh_ln_g,
           dh_ln_b, dh_out_w, dh_out_b):
    N = x.shape[0]
    E = edge_index.shape[1]
    u = jnp.nan_to_num(EigVecs)
    src = edge_index[0]
    dst = edge_index[1]

    mkey = jax.random.key(42)
    perm = jax.random.permutation(mkey, N)
    num_mask = int(MASK_RATIO * N)
    mask_nodes = perm[:num_mask]
    noise = NOISE_VAL * jax.random.normal(
        jax.random.fold_in(mkey, 1), (num_mask, u.shape[1]),
        dtype=jnp.float32)
    node_is_masked = jnp.zeros((N,), bool).at[mask_nodes].set(True)
    lut = jnp.full((N,), -1, jnp.int32).at[mask_nodes].set(
        jnp.arange(num_mask, dtype=jnp.int32))

    x0 = x[:, 0].astype(jnp.int32)

    wemb_pad = jnp.zeros((128, 128), jnp.float32).at[:NUM_ATOM_TYPE + 1].set(
        W_embed)
    hp_pad = wemb_pad @ W_pe2
    A = jnp.zeros((8, 128), jnp.float32).at[:4].set(W_edge @ W_pe2)
    b_row = w_pe @ W_pe2
    wedge_pad = jnp.zeros((8, 128), jnp.float32).at[:4].set(W_edge)
    mlm_wT_pad = jnp.zeros((128, 128), jnp.float32).at[:, :NUM_ATOM_TYPE].set(
        mlm_weight.T)
    mlm_bias_pad = jnp.zeros((1, 128), jnp.float32).at[0, :NUM_ATOM_TYPE].set(
        mlm_bias)

    noise_pad = jnp.zeros((N, u.shape[1]), jnp.float32).at[mask_nodes].add(
        noise)
    u_masked = u + noise_pad
    xm0 = jnp.where(node_is_masked, NUM_ATOM_TYPE, x0)
    atoms = xm0[src]
    du = u[src] - u[dst]
    pe2 = jnp.sum(du * du, axis=-1)
    dun = u_masked[src] - u_masked[dst]
    pn2 = jnp.sum(dun * dun, axis=-1)

    sdst = lut[dst]
    mm = sdst >= 0
    spread = jnp.arange(E, dtype=jnp.int32) % 8
    d1 = _NMP * 128
    safe1 = jnp.where(mm, sdst * 128 + atoms, d1 + spread)
    cg0 = jnp.zeros((d1 + 8,), jnp.float32).at[safe1].add(
        jnp.where(mm, 1.0, 0.0))[:d1].reshape(_NMP, 128)
    cells4 = (sdst[:, None] * 8 + jnp.arange(4, dtype=jnp.int32)).reshape(-1)
    mm4 = jnp.broadcast_to(mm[:, None], (E, 4)).reshape(-1)
    d2 = _NMP * 8
    safe4 = jnp.where(mm4, cells4,
                      d2 + jnp.arange(4 * E, dtype=jnp.int32) % 8)
    eam0 = jnp.zeros((d2 + 8,), jnp.float32).at[safe4].add(
        jnp.where(mm4, edge_attr.reshape(-1), 0.0))[:d2].reshape(_NMP, 8)
    safe2 = jnp.where(mm, sdst, _NMP + spread)
    pnm0 = jnp.zeros((_NMP + 8,), jnp.float32).at[safe2].add(
        jnp.where(mm, jnp.sqrt(pn2), 0.0))[:_NMP].reshape(_NMP, 1)
    cg = jnp.stack([cg0, jnp.zeros((_NMP, 128), jnp.float32)])
    eam = jnp.stack([eam0, jnp.zeros((_NMP, 8), jnp.float32)])
    pnm = jnp.stack([pnm0, jnp.zeros((_NMP, 1), jnp.float32)])

    snm = jnp.zeros((_NMP, 1), jnp.float32).at[:num_mask, 0].set(
        snorm_n[mask_nodes, 0])
    tgt = jnp.zeros((_NMP,), jnp.int32).at[:num_mask].set(x[mask_nodes, 0])

    v = _edge_chain(atoms, edge_attr, pe2, pn2, hp_pad, A, b_row, dh_dense_w,
                    dh_dense_b, dh_ln_g, dh_ln_b, dh_out_w, dh_out_b)[:, 0]
    atom_loss = _node_head(cg, eam, pnm, snm, wemb_pad, wedge_pad, w_pe,
                           W_gnn, mlm_dense_w, mlm_dense_b, mlm_ln_g,
                           mlm_ln_b, mlm_wT_pad, mlm_bias_pad, tgt)[0, 0]

    nd = _dedup_sc(src, dst, v, lut)
    num = jnp.sum(nd[:, :16])
    den = jnp.sum(nd[:, 16:])
    pe_loss = num / den

    return atom_loss + pe_loss
